# Initial kernel scaffold; baseline (speedup 1.0000x reference)
import jax
import jax.numpy as jnp
from jax.experimental import pallas as pl


def kernel(x, w1_aug, w_heads):
    raise NotImplementedError("write your pallas kernel here")


# speedup vs baseline: 13.4228x; 13.4228x over previous
import jax
import jax.numpy as jnp
from jax.experimental import pallas as pl
from jax.experimental.pallas import tpu as pltpu

_N = 8
_D = 32
_H = 32
_T = 4
_B_BLK = 1024


def _score_kernel(x_ref, w1a_ref, wh_ref, tr_ref, rot_ref, tor_ref):
    xv = x_ref[...].reshape(_B_BLK * _N, _D)
    w1 = w1a_ref[0:_D, :]
    b1 = w1a_ref[_D:_D + 1, :]
    h = jnp.dot(xv, w1, preferred_element_type=jnp.float32) + b1
    h = jnp.maximum(h, 0.0)
    pooled = jnp.sum(h.reshape(_B_BLK, _N, _H), axis=1)
    out = jnp.dot(pooled, wh_ref[...], preferred_element_type=jnp.float32)
    tr_ref[...] = out[:, 0:3]
    rot_ref[...] = out[:, 3:6]
    tor_ref[...] = out[:, 6:6 + _T]


@jax.jit
def _forward(x, w1_aug, w_heads):
    b = x.shape[0]
    n_blocks = pl.cdiv(b, _B_BLK)
    b_pad = n_blocks * _B_BLK
    if b_pad != b:
        x = jnp.pad(x, ((0, b_pad - b), (0, 0), (0, 0)))

    rows = b_pad * _N
    flops = 2 * rows * _D * _H + 2 * b_pad * _H * 128
    bytes_accessed = 4 * (rows * _D + (_D + 1) * _H + _H * 128 + b_pad * (3 + 3 + _T))

    tr, rot, tor = pl.pallas_call(
        _score_kernel,
        out_shape=[
            jax.ShapeDtypeStruct((b_pad, 3), jnp.float32),
            jax.ShapeDtypeStruct((b_pad, 3), jnp.float32),
            jax.ShapeDtypeStruct((b_pad, _T), jnp.float32),
        ],
        grid=(n_blocks,),
        in_specs=[
            pl.BlockSpec((_B_BLK, _N, _D), lambda i: (i, 0, 0)),
            pl.BlockSpec((_D + 1, _H), lambda i: (0, 0)),
            pl.BlockSpec((_H, 128), lambda i: (0, 0)),
        ],
        out_specs=[
            pl.BlockSpec((_B_BLK, 3), lambda i: (i, 0)),
            pl.BlockSpec((_B_BLK, 3), lambda i: (i, 0)),
            pl.BlockSpec((_B_BLK, _T), lambda i: (i, 0)),
        ],
        compiler_params=pltpu.CompilerParams(dimension_semantics=("parallel",)),
        cost_estimate=pl.CostEstimate(flops=flops, transcendentals=0,
                                      bytes_accessed=bytes_accessed),
    )(x, w1_aug, w_heads)

    return {"tr_pred": tr[:b], "rot_pred": rot[:b], "tor_pred": tor[:b]}


def kernel(x, w1_aug, w_heads):
    return _forward(x, w1_aug, w_heads)
